# baseline (speedup 1.0000x reference)
import jax
import jax.numpy as jnp
from jax import lax
from jax.experimental import pallas as pl
from jax.experimental.pallas import tpu as pltpu
from jax.experimental.pallas import tpu_sc as plsc

B, T = 4096, 50
D = 128
NBUF = 7


def kernel(x, embed_weight):
    info = plsc.get_sparse_core_info()
    nc, ns = info.num_cores, info.num_subcores
    nw = nc * ns
    per_w = B // nw

    mesh = plsc.VectorSubcoreMesh(core_axis_name="c", subcore_axis_name="s")

    @pl.kernel(
        out_type=jax.ShapeDtypeStruct((T, B, D), jnp.float32),
        mesh=mesh,
        scratch_types=[
            pltpu.VMEM((T, per_w), jnp.int32),
            pltpu.VMEM((NBUF, per_w, D), jnp.float32),
            pltpu.SemaphoreType.DMA((NBUF,)),
            pltpu.SemaphoreType.DMA((NBUF,)),
        ],
    )
    def run(xt_hbm, w_hbm, out_hbm, idx_v, rows_v, gsem, ssem):
        wid = lax.axis_index("s") * nc + lax.axis_index("c")
        b0 = wid * per_w
        pltpu.sync_copy(xt_hbm.at[:, pl.ds(b0, per_w)], idx_v)

        for b in range(NBUF):
            pltpu.async_copy(w_hbm.at[idx_v.at[b]], rows_v.at[b], gsem.at[b])

        K = 3

        def step(t, b):
            pltpu.make_async_copy(
                w_hbm.at[idx_v.at[0]], rows_v.at[b], gsem.at[b]).wait()
            pltpu.async_copy(
                rows_v.at[b], out_hbm.at[t].at[pl.ds(b0, per_w)], ssem.at[b])
            bp = (b - K) % NBUF
            tp = t - K
            @pl.when((tp >= 0) & (tp + NBUF < T))
            def _():
                pltpu.make_async_copy(
                    rows_v.at[bp], out_hbm.at[0].at[pl.ds(b0, per_w)],
                    ssem.at[bp]).wait()
                pltpu.async_copy(
                    w_hbm.at[idx_v.at[tp + NBUF]], rows_v.at[bp], gsem.at[bp])

        def outer(i, carry):
            for b in range(NBUF):
                step(i * NBUF + b, b)
            return carry

        n_full = T // NBUF
        lax.fori_loop(0, n_full, outer, 0)
        for b in range(T - n_full * NBUF):
            step(n_full * NBUF + b, b)

        for b in range(NBUF):
            pltpu.make_async_copy(
                rows_v.at[b], out_hbm.at[0].at[pl.ds(b0, per_w)],
                ssem.at[b]).wait()

    out_tr = run(x.astype(jnp.int32).T, embed_weight)
    return jnp.transpose(out_tr, (1, 0, 2))


# speedup vs baseline: 1.0102x; 1.0102x over previous
import jax
import jax.numpy as jnp
from jax import lax
from jax.experimental import pallas as pl
from jax.experimental.pallas import tpu as pltpu
from jax.experimental.pallas import tpu_sc as plsc

B, T = 4096, 50
D = 128
NBUF = 7


def kernel(x, embed_weight):
    info = plsc.get_sparse_core_info()
    nc, ns = info.num_cores, info.num_subcores
    nw = nc * ns
    per_w = B // nw

    mesh = plsc.VectorSubcoreMesh(core_axis_name="c", subcore_axis_name="s")

    @pl.kernel(
        out_type=jax.ShapeDtypeStruct((T, B, D), jnp.float32),
        mesh=mesh,
        scratch_types=[
            pltpu.VMEM((T, per_w), jnp.int32),
            pltpu.VMEM((NBUF, per_w, D), jnp.float32),
            pltpu.SemaphoreType.DMA((NBUF,)),
            pltpu.SemaphoreType.DMA((NBUF,)),
        ],
    )
    def run(xt_hbm, w_hbm, out_hbm, idx_v, rows_v, gsem, ssem):
        wid = lax.axis_index("s") * nc + lax.axis_index("c")
        b0 = wid * per_w
        pltpu.sync_copy(xt_hbm.at[:, pl.ds(b0, per_w)], idx_v)

        for b in range(NBUF):
            pltpu.async_copy(w_hbm.at[idx_v.at[b]], rows_v.at[b], gsem.at[b])

        def step(t, b):
            pltpu.make_async_copy(
                w_hbm.at[idx_v.at[0]], rows_v.at[b], gsem.at[b]).wait()
            pltpu.async_copy(
                rows_v.at[b], out_hbm.at[t].at[pl.ds(b0, per_w)], ssem.at[b])
            @pl.when(t + NBUF < T)
            def _():
                pltpu.make_async_copy(
                    rows_v.at[b], out_hbm.at[0].at[pl.ds(b0, per_w)],
                    ssem.at[b]).wait()
                pltpu.async_copy(
                    w_hbm.at[idx_v.at[t + NBUF]], rows_v.at[b], gsem.at[b])

        def outer(i, carry):
            for b in range(NBUF):
                step(i * NBUF + b, b)
            return carry

        n_full = T // NBUF
        lax.fori_loop(0, n_full, outer, 0)
        for b in range(T - n_full * NBUF):
            step(n_full * NBUF + b, b)

        for b in range(NBUF):
            pltpu.make_async_copy(
                rows_v.at[b], out_hbm.at[0].at[pl.ds(b0, per_w)],
                ssem.at[b]).wait()

    out_tr = run(x.astype(jnp.int32).T, embed_weight)
    return jnp.transpose(out_tr, (1, 0, 2))
